# baseline (speedup 1.0000x reference)
import functools

import jax
import jax.numpy as jnp
from jax import lax
from jax.experimental import pallas as pl
from jax.experimental.pallas import tpu as pltpu
from jax.experimental.pallas import tpu_sc as plsc

NW = 32
CHUNK = 128


def _gather_body(n_chunks, b_per_w, d,
                 idx_hbm, table_hbm, out_hbm,
                 idx_v, rows0, rows1, sem0, sem1):
    wid = lax.axis_index("s") * 2 + lax.axis_index("c")
    pltpu.sync_copy(idx_hbm.at[wid], idx_v)
    base = wid * b_per_w

    def body(k, carry):
        j0 = 2 * k
        j1 = 2 * k + 1
        cp0 = pltpu.async_copy(table_hbm.at[idx_v.at[j0]], rows0, sem0)
        cp1 = pltpu.async_copy(table_hbm.at[idx_v.at[j1]], rows1, sem1)
        cp0.wait()
        pltpu.sync_copy(rows0, out_hbm.at[pl.ds(base + j0 * CHUNK, CHUNK)])
        cp1.wait()
        pltpu.sync_copy(rows1, out_hbm.at[pl.ds(base + j1 * CHUNK, CHUNK)])
        return carry

    lax.fori_loop(0, n_chunks // 2, body, 0)


def kernel(paths, path_table):
    b, l = paths.shape
    v, d = path_table.shape
    n_flat = b * l
    b_per_w = n_flat // NW
    n_chunks = b_per_w // CHUNK

    idx = paths.reshape(NW, n_chunks, CHUNK).astype(jnp.int32)

    mesh = plsc.VectorSubcoreMesh(core_axis_name="c", subcore_axis_name="s")
    grid_kernel = functools.partial(
        pl.kernel,
        out_type=jax.ShapeDtypeStruct((n_flat, d), jnp.float32),
        mesh=mesh,
        compiler_params=pltpu.CompilerParams(use_tc_tiling_on_sc=False),
        scratch_types=[
            pltpu.VMEM((n_chunks, CHUNK), jnp.int32),
            pltpu.VMEM((CHUNK, d), jnp.float32),
            pltpu.VMEM((CHUNK, d), jnp.float32),
            pltpu.SemaphoreType.DMA,
            pltpu.SemaphoreType.DMA,
        ],
    )(functools.partial(_gather_body, n_chunks, b_per_w, d))

    out = grid_kernel(idx, path_table)
    return out.reshape(b, l, d)


# speedup vs baseline: 1.0488x; 1.0488x over previous
import functools

import jax
import jax.numpy as jnp
from jax import lax
from jax.experimental import pallas as pl
from jax.experimental.pallas import tpu as pltpu
from jax.experimental.pallas import tpu_sc as plsc

NW = 32
CHUNK = 128
NBUF = 8


def _gather_body(n_chunks, b_per_w, d,
                 idx_hbm, table_hbm, out_hbm, idx_v, rows, sems):
    wid = lax.axis_index("s") * 2 + lax.axis_index("c")
    pltpu.sync_copy(idx_hbm.at[wid], idx_v)
    base = wid * b_per_w

    def g_start(j, b):
        pltpu.async_copy(table_hbm.at[idx_v.at[j]], rows[b], sems[b])

    def s_start(j, b):
        pltpu.async_copy(rows[b], out_hbm.at[pl.ds(base + j * CHUNK, CHUNK)],
                         sems[b])

    def wait(b):
        pltpu.make_async_copy(out_hbm.at[pl.ds(0, CHUNK)], rows[b],
                              sems[b]).wait()

    half = NBUF // 2

    for b in range(half):
        g_start(b, b)
    for j in range(half):
        g_wait_b = j
        wait(g_wait_b)
        s_start(j, j)
        g_start(j + half, j + half)
    for j in range(half, NBUF):
        wait(j - half)
        g_start(j + half, j - half)
        wait(j)
        s_start(j, j)

    def body(k, carry):
        j0 = k * NBUF
        for b in range(NBUF):
            j = j0 + b
            wait((b + half) % NBUF)
            g_start(j + half, (b + half) % NBUF)
            wait(b)
            s_start(j, b)
        return carry

    lax.fori_loop(1, n_chunks // NBUF - 1, body, 0)

    j0 = n_chunks - NBUF
    for b in range(half):
        j = j0 + b
        wait(b + half)
        g_start(j + half, b + half)
        wait(b)
        s_start(j, b)
    for b in range(half, NBUF):
        j = j0 + b
        wait(b - half)
        wait(b)
        s_start(j, b)
    for b in range(half, NBUF):
        wait(b)


def kernel(paths, path_table):
    b, l = paths.shape
    v, d = path_table.shape
    n_flat = b * l
    b_per_w = n_flat // NW
    n_chunks = b_per_w // CHUNK

    idx = paths.reshape(NW, n_chunks, CHUNK).astype(jnp.int32)

    mesh = plsc.VectorSubcoreMesh(core_axis_name="c", subcore_axis_name="s")
    grid_kernel = functools.partial(
        pl.kernel,
        out_type=jax.ShapeDtypeStruct((n_flat, d), jnp.float32),
        mesh=mesh,
        compiler_params=pltpu.CompilerParams(use_tc_tiling_on_sc=False),
        scratch_types=[
            pltpu.VMEM((n_chunks, CHUNK), jnp.int32),
            [pltpu.VMEM((CHUNK, d), jnp.float32) for _ in range(NBUF)],
            [pltpu.SemaphoreType.DMA for _ in range(NBUF)],
        ],
    )(functools.partial(_gather_body, n_chunks, b_per_w, d))

    out = grid_kernel(idx, path_table)
    return out.reshape(b, l, d)


# speedup vs baseline: 1.4940x; 1.4245x over previous
import functools

import jax
import jax.numpy as jnp
from jax import lax
from jax.experimental import pallas as pl
from jax.experimental.pallas import tpu as pltpu
from jax.experimental.pallas import tpu_sc as plsc

NW = 32
CHUNK = 128
NBUF = 8


def _gather_body(n_chunks, b_per_w, d,
                 idx_hbm, table_hbm, out_hbm, idx_v, rows, sems):
    wid = lax.axis_index("s") * 2 + lax.axis_index("c")
    pltpu.sync_copy(idx_hbm.at[wid], idx_v)
    base = wid * b_per_w

    def g_start(j, b):
        pltpu.async_copy(table_hbm.at[idx_v.at[j]], rows[b], sems[b])

    def s_start(j, b):
        pltpu.async_copy(
            rows[b],
            out_hbm.at[pl.ds(base + j * CHUNK, CHUNK), pl.ds(0, d)],
            sems[b])

    def wait(b):
        pltpu.make_async_copy(out_hbm.at[pl.ds(0, CHUNK), pl.ds(0, d)],
                              rows[b], sems[b]).wait()

    half = NBUF // 2
    for b in range(half):
        g_start(b, b)
    for j in range(half):
        wait(j)
        s_start(j, j)
        g_start(j + half, j + half)
    for j in range(half, NBUF):
        wait(j - half)
        g_start(j + half, j - half)
        wait(j)
        s_start(j, j)

    def body(k, carry):
        j0 = k * NBUF
        for b in range(NBUF):
            j = j0 + b
            wait((b + half) % NBUF)
            g_start(j + half, (b + half) % NBUF)
            wait(b)
            s_start(j, b)
        return carry

    lax.fori_loop(1, n_chunks // NBUF - 1, body, 0)

    j0 = n_chunks - NBUF
    for b in range(half):
        j = j0 + b
        wait(b + half)
        g_start(j + half, b + half)
        wait(b)
        s_start(j, b)
    for b in range(half, NBUF):
        j = j0 + b
        wait(b - half)
        wait(b)
        s_start(j, b)
    for b in range(half, NBUF):
        wait(b)


def kernel(paths, path_table):
    b, l = paths.shape
    v, d = path_table.shape
    n_flat = b * l
    b_per_w = n_flat // NW
    n_chunks = b_per_w // CHUNK

    idx = (paths * 2).reshape(NW, n_chunks, CHUNK).astype(jnp.int32)
    tbl2 = jnp.pad(path_table, ((0, 0), (0, d))).reshape(2 * v, d)

    mesh = plsc.VectorSubcoreMesh(core_axis_name="c", subcore_axis_name="s")
    grid_kernel = functools.partial(
        pl.kernel,
        out_type=jax.ShapeDtypeStruct((n_flat, 2 * d), jnp.float32),
        mesh=mesh,
        compiler_params=pltpu.CompilerParams(use_tc_tiling_on_sc=False),
        scratch_types=[
            pltpu.VMEM((n_chunks, CHUNK), jnp.int32),
            [pltpu.VMEM((CHUNK, d), jnp.float32) for _ in range(NBUF)],
            [pltpu.SemaphoreType.DMA for _ in range(NBUF)],
        ],
    )(functools.partial(_gather_body, n_chunks, b_per_w, d))

    out = grid_kernel(idx, tbl2)
    return out.reshape(b, l, 2 * d)[:, :, :d]
